# Initial kernel scaffold; baseline (speedup 1.0000x reference)
import jax
import jax.numpy as jnp
from jax.experimental import pallas as pl


def kernel(inputs, embeddings):
    raise NotImplementedError("write your pallas kernel here")


# speedup vs baseline: 1.1029x; 1.1029x over previous
import functools

import jax
import jax.numpy as jnp
from jax import lax
from jax.experimental import pallas as pl
from jax.experimental.pallas import tpu as pltpu
from jax.experimental.pallas import tpu_sc as plsc

VOCAB = 1000000
EMBED_DIM = 32
BATCH = 16384
HIST = 50

_NC = 2
_NS = 16
_NW = _NC * _NS

_B = BATCH * HIST
_B_PER_W = _B // _NW
_CHUNK = 1600
_N_CHUNKS = _B_PER_W // _CHUNK


def _gather_kernel(table_hbm, idx_hbm, out_hbm, idx_v, rows_v, sem):
    wid = lax.axis_index("s") * _NC + lax.axis_index("c")
    base = wid * _B_PER_W

    def body(c, carry):
        start = base + c * _CHUNK
        pltpu.sync_copy(idx_hbm.at[pl.ds(start, _CHUNK)], idx_v)
        pltpu.async_copy(table_hbm.at[idx_v], rows_v, sem).wait()
        pltpu.sync_copy(rows_v, out_hbm.at[pl.ds(start, _CHUNK)])
        return carry

    lax.fori_loop(0, _N_CHUNKS, body, 0)


@jax.jit
def _embedding_lookup(idx_flat, embeddings):
    mesh = plsc.VectorSubcoreMesh(core_axis_name="c", subcore_axis_name="s")
    k = functools.partial(
        pl.kernel,
        mesh=mesh,
        out_type=jax.ShapeDtypeStruct((_B, EMBED_DIM), jnp.float32),
        scratch_types=[
            pltpu.VMEM((_CHUNK,), jnp.int32),
            pltpu.VMEM((_CHUNK, EMBED_DIM), jnp.float32),
            pltpu.SemaphoreType.DMA,
        ],
        compiler_params=pltpu.CompilerParams(use_tc_tiling_on_sc=False),
    )(_gather_kernel)
    return k(embeddings, idx_flat)


def kernel(inputs, embeddings):
    idx_flat = inputs.astype(jnp.int32).reshape(_B)
    out = _embedding_lookup(idx_flat, embeddings)
    return out.reshape(BATCH, HIST, EMBED_DIM)


# speedup vs baseline: 1.1128x; 1.0090x over previous
import functools

import jax
import jax.numpy as jnp
from jax import lax
from jax.experimental import pallas as pl
from jax.experimental.pallas import tpu as pltpu
from jax.experimental.pallas import tpu_sc as plsc

VOCAB = 1000000
EMBED_DIM = 32
BATCH = 16384
HIST = 50

_NC = 2
_NS = 16
_NW = _NC * _NS

_B = BATCH * HIST
_B_PER_W = _B // _NW
_CHUNK = 1600
_N_CHUNKS = _B_PER_W // _CHUNK


def _gather_kernel(table_hbm, idx_hbm, out_hbm,
                   idx_v, rows0, rows1, gsem0, gsem1, wsem0, wsem1):
    wid = lax.axis_index("s") * _NC + lax.axis_index("c")
    base = wid * _B_PER_W

    rows = (rows0, rows1)
    gsem = (gsem0, gsem1)
    wsem = (wsem0, wsem1)

    def idx_load(c):
        pltpu.sync_copy(idx_hbm.at[pl.ds(base + c * _CHUNK, _CHUNK)],
                        idx_v.at[c % 4])

    gh = {}
    wh = {}

    idx_load(0)
    for c in range(_N_CHUNKS):
        b = c % 2
        if c >= 2:
            wh[c - 2].wait()
        gh[c] = pltpu.async_copy(table_hbm.at[idx_v.at[c % 4]], rows[b],
                                 gsem[b])
        if c + 1 < _N_CHUNKS:
            idx_load(c + 1)
        if c >= 1:
            gh[c - 1].wait()
            pb = (c - 1) % 2
            wh[c - 1] = pltpu.async_copy(
                rows[pb],
                out_hbm.at[pl.ds(base + (c - 1) * _CHUNK, _CHUNK)],
                wsem[pb])
    c = _N_CHUNKS - 1
    gh[c].wait()
    wh[c] = pltpu.async_copy(
        rows[c % 2], out_hbm.at[pl.ds(base + c * _CHUNK, _CHUNK)],
        wsem[c % 2])
    wh[c - 1].wait()
    wh[c].wait()


@jax.jit
def _embedding_lookup(idx_flat, embeddings):
    mesh = plsc.VectorSubcoreMesh(core_axis_name="c", subcore_axis_name="s")
    k = functools.partial(
        pl.kernel,
        mesh=mesh,
        out_type=jax.ShapeDtypeStruct((_B, EMBED_DIM), jnp.float32),
        scratch_types=[
            pltpu.VMEM((4, _CHUNK), jnp.int32),
            pltpu.VMEM((_CHUNK, EMBED_DIM), jnp.float32),
            pltpu.VMEM((_CHUNK, EMBED_DIM), jnp.float32),
            pltpu.SemaphoreType.DMA,
            pltpu.SemaphoreType.DMA,
            pltpu.SemaphoreType.DMA,
            pltpu.SemaphoreType.DMA,
        ],
        compiler_params=pltpu.CompilerParams(use_tc_tiling_on_sc=False),
    )(_gather_kernel)
    return k(embeddings, idx_flat)


def kernel(inputs, embeddings):
    idx_flat = inputs.astype(jnp.int32).reshape(_B)
    out = _embedding_lookup(idx_flat, embeddings)
    return out.reshape(BATCH, HIST, EMBED_DIM)


# speedup vs baseline: 1.1131x; 1.0003x over previous
import functools

import jax
import jax.numpy as jnp
from jax import lax
from jax.experimental import pallas as pl
from jax.experimental.pallas import tpu as pltpu
from jax.experimental.pallas import tpu_sc as plsc

VOCAB = 1000000
EMBED_DIM = 32
BATCH = 16384
HIST = 50

_NC = 2
_NS = 16
_NW = _NC * _NS

_B = BATCH * HIST
_B_PER_W = _B // _NW
_CHUNK = 1600
_N_CHUNKS = _B_PER_W // _CHUNK
_NSUB = 4


def _gather_kernel(table_hbm, idx_hbm, out_hbm,
                   idx_v, rows0, rows1, gsem0, gsem1, wsem0, wsem1):
    wid = lax.axis_index("s") * _NC + lax.axis_index("c")
    base = wid * _B_PER_W

    rows = (rows0, rows1)
    gsem = (gsem0, gsem1)
    wsem = (wsem0, wsem1)

    def idx_load(c):
        pltpu.sync_copy(idx_hbm.at[pl.ds(base + c * _CHUNK, _CHUNK)],
                        idx_v.at[c % 4])

    gh = {}
    wh = {}

    idx_load(0)
    for c in range(_N_CHUNKS):
        b = c % 2
        if c >= 2:
            wh[c - 2].wait()
        sub = _CHUNK // _NSUB
        gh[c] = [
            pltpu.async_copy(
                table_hbm.at[idx_v.at[c % 4, pl.ds(s * sub, sub)]],
                rows[b].at[pl.ds(s * sub, sub)],
                gsem[b])
            for s in range(_NSUB)
        ]
        if c + 1 < _N_CHUNKS:
            idx_load(c + 1)
        if c >= 1:
            for h in gh[c - 1]:
                h.wait()
            pb = (c - 1) % 2
            wh[c - 1] = pltpu.async_copy(
                rows[pb],
                out_hbm.at[pl.ds(base + (c - 1) * _CHUNK, _CHUNK)],
                wsem[pb])
    c = _N_CHUNKS - 1
    for h in gh[c]:
        h.wait()
    wh[c] = pltpu.async_copy(
        rows[c % 2], out_hbm.at[pl.ds(base + c * _CHUNK, _CHUNK)],
        wsem[c % 2])
    wh[c - 1].wait()
    wh[c].wait()


@jax.jit
def _embedding_lookup(idx_flat, embeddings):
    mesh = plsc.VectorSubcoreMesh(core_axis_name="c", subcore_axis_name="s")
    k = functools.partial(
        pl.kernel,
        mesh=mesh,
        out_type=jax.ShapeDtypeStruct((_B, EMBED_DIM), jnp.float32),
        scratch_types=[
            pltpu.VMEM((4, _CHUNK), jnp.int32),
            pltpu.VMEM((_CHUNK, EMBED_DIM), jnp.float32),
            pltpu.VMEM((_CHUNK, EMBED_DIM), jnp.float32),
            pltpu.SemaphoreType.DMA,
            pltpu.SemaphoreType.DMA,
            pltpu.SemaphoreType.DMA,
            pltpu.SemaphoreType.DMA,
        ],
        compiler_params=pltpu.CompilerParams(use_tc_tiling_on_sc=False),
    )(_gather_kernel)
    return k(embeddings, idx_flat)


def kernel(inputs, embeddings):
    idx_flat = inputs.astype(jnp.int32).reshape(_B)
    out = _embedding_lookup(idx_flat, embeddings)
    return out.reshape(BATCH, HIST, EMBED_DIM)


# speedup vs baseline: 1.3897x; 1.2484x over previous
import functools

import jax
import jax.numpy as jnp
from jax import lax
from jax.experimental import pallas as pl
from jax.experimental.pallas import tpu as pltpu
from jax.experimental.pallas import tpu_sc as plsc

VOCAB = 1000000
EMBED_DIM = 32
BATCH = 16384
HIST = 50

_NC = 2
_NS = 16
_NW = _NC * _NS

_B = BATCH * HIST
_BPW = BATCH // _NW
_BB = 128
_HC = 5
_NBB = _BPW // _BB
_NHC = HIST // _HC
_CH = _HC * _BB


def _gather_kernel(tbl_hbm, idx_hbm, out_hbm, idx_w, q_v, r_v, g_v, s_v, gsem):
    wid = lax.axis_index("s") * _NC + lax.axis_index("c")
    b0 = wid * _BPW
    lane = lax.iota(jnp.int32, 16)

    pltpu.sync_copy(idx_hbm.at[pl.ds(b0 * HIST, _BPW * HIST)], idx_w)

    def chunk(t, carry):
        bb = t // _NHC
        h0 = (t % _NHC) * _HC

        for h in range(_HC):
            for blk in range(_BB // 16):
                j = (bb * _BB + blk * 16 + lane) * HIST + (h0 + h)
                v = plsc.load_gather(idx_w, [j])
                q_v[pl.ds(h * _BB + blk * 16, 16)] = v >> 2
                r_v[pl.ds(h * _BB + blk * 16, 16)] = (v & 3) * EMBED_DIM

        pltpu.async_copy(tbl_hbm.at[q_v], g_v, gsem).wait()

        for h in range(_HC):
            for blk in range(_BB // 16):
                row = h * _BB + blk * 16 + lane
                colb = r_v[pl.ds(h * _BB + blk * 16, 16)]
                for e in range(EMBED_DIM):
                    s_v[h, e, pl.ds(blk * 16, 16)] = plsc.load_gather(
                        g_v, [row, colb + e])

        pltpu.sync_copy(
            s_v, out_hbm.at[pl.ds(h0, _HC), :, pl.ds(b0 + bb * _BB, _BB)])
        return carry

    lax.fori_loop(0, _NBB * _NHC, chunk, 0)


@jax.jit
def _embedding_lookup(idx_flat, tbl4):
    mesh = plsc.VectorSubcoreMesh(core_axis_name="c", subcore_axis_name="s")
    k = functools.partial(
        pl.kernel,
        mesh=mesh,
        out_type=jax.ShapeDtypeStruct((HIST, EMBED_DIM, BATCH), jnp.float32),
        scratch_types=[
            pltpu.VMEM((_BPW * HIST,), jnp.int32),
            pltpu.VMEM((_CH,), jnp.int32),
            pltpu.VMEM((_CH,), jnp.int32),
            pltpu.VMEM((_CH, 128), jnp.float32),
            pltpu.VMEM((_HC, EMBED_DIM, _BB), jnp.float32),
            pltpu.SemaphoreType.DMA,
        ],
        compiler_params=pltpu.CompilerParams(needs_layout_passes=False),
    )(_gather_kernel)
    return k(tbl4, idx_flat)


def kernel(inputs, embeddings):
    idx_flat = inputs.astype(jnp.int32).reshape(_B)
    tbl4 = embeddings.reshape(VOCAB // 4, EMBED_DIM * 4)
    out_t = _embedding_lookup(idx_flat, tbl4)
    return out_t.transpose(2, 0, 1)
